# Initial kernel scaffold; baseline (speedup 1.0000x reference)
import jax
import jax.numpy as jnp
from jax.experimental import pallas as pl


def kernel(x, edge_index_r0, edge_index_r1, W0_r0, b0_r0, W0_r1, b0_r1, W1_r0, b1_r0, W1_r1, b1_r1):
    raise NotImplementedError("write your pallas kernel here")


# speedup vs baseline: 7.1037x; 7.1037x over previous
import jax
import jax.numpy as jnp
from jax import lax
from jax.experimental import pallas as pl
from jax.experimental.pallas import tpu as pltpu
from jax.experimental.pallas import tpu_sc as plsc

N = 10000
D = 128
E = 160000

NS = 16
CH = 125
CROWS = E // CH
CROW_PT = CROWS // NS
WAVE = 8
HALF = 2
IT_PT = CROW_PT // WAVE
NPAD = NS * 640
ZCH = 80
NZCH = N // ZCH

_mesh = plsc.VectorSubcoreMesh(core_axis_name="c", subcore_axis_name="s")


def _deg_body(e0, e1, z_n, og0, ig0, og1, ig1, da, db, idx_v, ones_v, ssem):
    c = lax.axis_index("c")
    t = lax.axis_index("s")
    for j in range(128 // 16):
        ones_v[pl.ds(j * 16, 16)] = jnp.full((16,), 1.0, jnp.float32)
    own = pl.ds(t * 640, 640)
    pltpu.sync_copy(z_n.at[own], da.at[own])
    pltpu.sync_copy(z_n.at[own], db.at[own])
    plsc.subcore_barrier()
    ones_sl = ones_v.at[pl.ds(0, CH)]

    def run(e):
        @pl.loop(0, IT_PT)
        def body(i):
            rb = t * CROW_PT + i * WAVE
            pltpu.sync_copy(e.at[0, pl.ds(rb, WAVE)], idx_v.at[0])
            pltpu.sync_copy(e.at[1, pl.ds(rb, WAVE)], idx_v.at[1])
            hs = [pltpu.async_copy(ones_sl, da.at[idx_v.at[0, j]], ssem, add=True)
                  for j in range(WAVE)]
            hs += [pltpu.async_copy(ones_sl, db.at[idx_v.at[1, j]], ssem, add=True)
                   for j in range(WAVE)]
            for h in hs:
                h.wait()

    pl.when(c == 0)(lambda: run(e0))
    pl.when(c == 1)(lambda: run(e1))
    plsc.subcore_barrier()

    def wb(og, ig):
        pltpu.sync_copy(da.at[own], og.at[own])
        pltpu.sync_copy(db.at[own], ig.at[own])

    pl.when(c == 0)(lambda: wb(og0, ig0))
    pl.when(c == 1)(lambda: wb(og1, ig1))


_deg_call = pl.kernel(
    _deg_body,
    out_type=(jax.ShapeDtypeStruct((NPAD,), jnp.float32),
              jax.ShapeDtypeStruct((NPAD,), jnp.float32),
              jax.ShapeDtypeStruct((NPAD,), jnp.float32),
              jax.ShapeDtypeStruct((NPAD,), jnp.float32)),
    mesh=_mesh,
    scratch_types=[
        pltpu.VMEM_SHARED((NPAD,), jnp.float32),
        pltpu.VMEM_SHARED((NPAD,), jnp.float32),
        pltpu.VMEM((2, WAVE, CH), jnp.int32),
        pltpu.VMEM((128,), jnp.float32),
        pltpu.SemaphoreType.DMA,
    ],
)


def _agg_body(xs0, xs1, e0, e1, znd, y0, y1, acc, idx_v, rows_v, gsem, ssem):
    c = lax.axis_index("c")
    t = lax.axis_index("s")
    for i in range(8):
        ch = t + NS * i

        @pl.when(ch < NZCH)
        def _():
            sl = pl.ds(ch * ZCH, ZCH)
            pltpu.sync_copy(znd.at[sl], acc.at[sl])

    plsc.subcore_barrier()

    def run(xs, e):
        @pl.loop(0, IT_PT)
        def body(i):
            rb = t * CROW_PT + i * WAVE
            pltpu.sync_copy(e.at[0, pl.ds(rb, WAVE)], idx_v.at[0])
            pltpu.sync_copy(e.at[1, pl.ds(rb, WAVE)], idx_v.at[1])
            for half in range(WAVE // HALF):
                gs = [pltpu.async_copy(xs.at[idx_v.at[0, half * HALF + q]],
                                       rows_v.at[q], gsem)
                      for q in range(HALF)]
                for h in gs:
                    h.wait()
                ss = [pltpu.async_copy(rows_v.at[q],
                                       acc.at[idx_v.at[1, half * HALF + q]],
                                       ssem, add=True)
                      for q in range(HALF)]
                for h in ss:
                    h.wait()

    pl.when(c == 0)(lambda: run(xs0, e0))
    pl.when(c == 1)(lambda: run(xs1, e1))
    plsc.subcore_barrier()

    def wb(y):
        for i in range(8):
            ch = t + NS * i

            @pl.when(ch < NZCH)
            def _():
                sl = pl.ds(ch * ZCH, ZCH)
                pltpu.sync_copy(acc.at[sl], y.at[sl])

    pl.when(c == 0)(lambda: wb(y0))
    pl.when(c == 1)(lambda: wb(y1))


_agg_call = pl.kernel(
    _agg_body,
    out_type=(jax.ShapeDtypeStruct((N, D), jnp.float32),
              jax.ShapeDtypeStruct((N, D), jnp.float32)),
    mesh=_mesh,
    scratch_types=[
        pltpu.VMEM_SHARED((N, D), jnp.float32),
        pltpu.VMEM((2, WAVE, CH), jnp.int32),
        pltpu.VMEM((HALF, CH, D), jnp.float32),
        pltpu.SemaphoreType.DMA,
        pltpu.SemaphoreType.DMA,
    ],
)


def _nrm(d):
    return jnp.where(d > 0, lax.rsqrt(jnp.maximum(d, 1.0)), 0.0)


def _norm_body(x_ref, od0, id0, od1, id1, xs0, xs1, ns0, ns1, nd0, nd1):
    a = _nrm(od0[...])
    b = _nrm(od1[...])
    xv = x_ref[...]
    xs0[...] = xv * a
    xs1[...] = xv * b
    ns0[...] = a
    ns1[...] = b
    nd0[...] = _nrm(id0[...])
    nd1[...] = _nrm(id1[...])


_BLK = 1000
_vec_spec = pl.BlockSpec((_BLK, 1), lambda i: (i, 0))
_mat_spec = pl.BlockSpec((_BLK, D), lambda i: (i, 0))

_norm_call = pl.pallas_call(
    _norm_body,
    grid=(N // _BLK,),
    in_specs=[_mat_spec, _vec_spec, _vec_spec, _vec_spec, _vec_spec],
    out_specs=[_mat_spec, _mat_spec, _vec_spec, _vec_spec, _vec_spec, _vec_spec],
    out_shape=[jax.ShapeDtypeStruct((N, D), jnp.float32),
               jax.ShapeDtypeStruct((N, D), jnp.float32),
               jax.ShapeDtypeStruct((N, 1), jnp.float32),
               jax.ShapeDtypeStruct((N, 1), jnp.float32),
               jax.ShapeDtypeStruct((N, 1), jnp.float32),
               jax.ShapeDtypeStruct((N, 1), jnp.float32)],
)


def _w_spec():
    return pl.BlockSpec((D, D), lambda i: (0, 0))


def _comb0_body(y0, y1, nd0, nd1, ns0, ns1, w0, w1, bs, hs0, hs1):
    h = jnp.dot(y0[...] * nd0[...], w0[...], preferred_element_type=jnp.float32)
    h = h + jnp.dot(y1[...] * nd1[...], w1[...], preferred_element_type=jnp.float32)
    h = jnp.maximum(h + bs[...], 0.0)
    hs0[...] = h * ns0[...]
    hs1[...] = h * ns1[...]


_comb0_call = pl.pallas_call(
    _comb0_body,
    grid=(N // _BLK,),
    in_specs=[_mat_spec, _mat_spec, _vec_spec, _vec_spec, _vec_spec, _vec_spec,
              _w_spec(), _w_spec(), pl.BlockSpec((1, D), lambda i: (0, 0))],
    out_specs=[_mat_spec, _mat_spec],
    out_shape=[jax.ShapeDtypeStruct((N, D), jnp.float32),
               jax.ShapeDtypeStruct((N, D), jnp.float32)],
)


def _comb1_body(y0, y1, nd0, nd1, w0, w1, bs, out):
    h = jnp.dot(y0[...] * nd0[...], w0[...], preferred_element_type=jnp.float32)
    h = h + jnp.dot(y1[...] * nd1[...], w1[...], preferred_element_type=jnp.float32)
    out[...] = h + bs[...]


_comb1_call = pl.pallas_call(
    _comb1_body,
    grid=(N // _BLK,),
    in_specs=[_mat_spec, _mat_spec, _vec_spec, _vec_spec,
              _w_spec(), _w_spec(), pl.BlockSpec((1, D), lambda i: (0, 0))],
    out_specs=_mat_spec,
    out_shape=jax.ShapeDtypeStruct((N, D), jnp.float32),
)


@jax.jit
def kernel(x, edge_index_r0, edge_index_r1, W0_r0, b0_r0, W0_r1, b0_r1,
           W1_r0, b1_r0, W1_r1, b1_r1):
    e0 = edge_index_r0.reshape(2, CROWS, CH)
    e1 = edge_index_r1.reshape(2, CROWS, CH)
    z_n = jnp.zeros((NPAD,), jnp.float32)
    znd = jnp.zeros((N, D), jnp.float32)

    og0, ig0, og1, ig1 = _deg_call(e0, e1, z_n)
    od0 = og0[:N].reshape(N, 1)
    id0 = ig0[:N].reshape(N, 1)
    od1 = og1[:N].reshape(N, 1)
    id1 = ig1[:N].reshape(N, 1)

    xs0, xs1, ns0, ns1, nd0, nd1 = _norm_call(x, od0, id0, od1, id1)
    y00, y01 = _agg_call(xs0, xs1, e0, e1, znd)
    bs0 = (b0_r0 + b0_r1).reshape(1, D)
    hs0, hs1 = _comb0_call(y00, y01, nd0, nd1, ns0, ns1, W0_r0, W0_r1, bs0)
    y10, y11 = _agg_call(hs0, hs1, e0, e1, znd)
    bs1 = (b1_r0 + b1_r1).reshape(1, D)
    return _comb1_call(y10, y11, nd0, nd1, W1_r0, W1_r1, bs1)


# speedup vs baseline: 8.1714x; 1.1503x over previous
import jax
import jax.numpy as jnp
from jax import lax
from jax.experimental import pallas as pl
from jax.experimental.pallas import tpu as pltpu
from jax.experimental.pallas import tpu_sc as plsc

N = 10000
D = 128
E = 160000

NS = 16
CH = 125
CROWS = E // CH
CROW_PT = CROWS // NS
WAVE = 8
IT_PT = CROW_PT // WAVE
NPAD = NS * 640
ZCH = 80
NZCH = N // ZCH

_mesh = plsc.VectorSubcoreMesh(core_axis_name="c", subcore_axis_name="s")


def _deg_body(e0, e1, z_n, og0, ig0, og1, ig1, da, db, idx_v, ones_v, ssem):
    c = lax.axis_index("c")
    t = lax.axis_index("s")
    for j in range(128 // 16):
        ones_v[pl.ds(j * 16, 16)] = jnp.full((16,), 1.0, jnp.float32)
    own = pl.ds(t * 640, 640)
    pltpu.sync_copy(z_n.at[own], da.at[own])
    pltpu.sync_copy(z_n.at[own], db.at[own])
    plsc.subcore_barrier()
    ones_sl = ones_v.at[pl.ds(0, CH)]

    def run(e):
        @pl.loop(0, IT_PT)
        def body(i):
            rb = t * CROW_PT + i * WAVE
            pltpu.sync_copy(e.at[0, pl.ds(rb, WAVE)], idx_v.at[0])
            pltpu.sync_copy(e.at[1, pl.ds(rb, WAVE)], idx_v.at[1])
            hs = [pltpu.async_copy(ones_sl, da.at[idx_v.at[0, j]], ssem, add=True)
                  for j in range(WAVE)]
            hs += [pltpu.async_copy(ones_sl, db.at[idx_v.at[1, j]], ssem, add=True)
                   for j in range(WAVE)]
            for h in hs:
                h.wait()

    pl.when(c == 0)(lambda: run(e0))
    pl.when(c == 1)(lambda: run(e1))
    plsc.subcore_barrier()

    def wb(og, ig):
        pltpu.sync_copy(da.at[own], og.at[own])
        pltpu.sync_copy(db.at[own], ig.at[own])

    pl.when(c == 0)(lambda: wb(og0, ig0))
    pl.when(c == 1)(lambda: wb(og1, ig1))


_deg_call = pl.kernel(
    _deg_body,
    out_type=(jax.ShapeDtypeStruct((NPAD,), jnp.float32),
              jax.ShapeDtypeStruct((NPAD,), jnp.float32),
              jax.ShapeDtypeStruct((NPAD,), jnp.float32),
              jax.ShapeDtypeStruct((NPAD,), jnp.float32)),
    mesh=_mesh,
    scratch_types=[
        pltpu.VMEM_SHARED((NPAD,), jnp.float32),
        pltpu.VMEM_SHARED((NPAD,), jnp.float32),
        pltpu.VMEM((2, WAVE, CH), jnp.int32),
        pltpu.VMEM((128,), jnp.float32),
        pltpu.SemaphoreType.DMA,
    ],
)


def _maybe(pred, fn):
    def run():
        fn()

    if pred is None:
        run()
    else:
        pl.when(pred)(run)


def _agg_body(xs0, xs1, e0, e1, znd, y0, y1, acc, idx_v, rows_v, gsem, isem):
    c = lax.axis_index("c")
    t = lax.axis_index("s")
    for i in range(8):
        ch = t + NS * i

        @pl.when(ch < NZCH)
        def _():
            sl = pl.ds(ch * ZCH, ZCH)
            pltpu.sync_copy(znd.at[sl], acc.at[sl])

    plsc.subcore_barrier()
    base = t * CROW_PT

    def run(xs, e):
        pltpu.sync_copy(e.at[:, pl.ds(base, WAVE)], idx_v.at[0])
        pltpu.async_copy(xs.at[idx_v.at[0, 0, 0]], rows_v.at[0], gsem)

        def wave(w, pw, pre_pred, next_pred):
            po = 1 - pw
            for q in range(WAVE):
                pltpu.make_async_copy(xs.at[idx_v.at[pw, 0, q]],
                                      rows_v.at[q % 2], gsem).wait()
                if q == 0:
                    _maybe(pre_pred, lambda: pltpu.async_copy(
                        e.at[:, pl.ds(base + (w + 1) * WAVE, WAVE)],
                        idx_v.at[po], isem))
                if q < WAVE - 1:
                    pltpu.async_copy(xs.at[idx_v.at[pw, 0, q + 1]],
                                     rows_v.at[(q + 1) % 2], gsem)
                if q == WAVE - 2:
                    _maybe(pre_pred, lambda: pltpu.make_async_copy(
                        e.at[:, pl.ds(base, WAVE)], idx_v.at[po], isem).wait())
                if q == WAVE - 1:
                    _maybe(next_pred, lambda: pltpu.async_copy(
                        xs.at[idx_v.at[po, 0, 0]], rows_v.at[0], gsem))
                pltpu.sync_copy(rows_v.at[q % 2], acc.at[idx_v.at[pw, 1, q]],
                                add=True)

        @pl.loop(0, IT_PT // 2)
        def pair(i):
            not_last = i < IT_PT // 2 - 1
            wave(2 * i, 0, None, None)
            wave(2 * i + 1, 1, not_last, not_last)

    pl.when(c == 0)(lambda: run(xs0, e0))
    pl.when(c == 1)(lambda: run(xs1, e1))
    plsc.subcore_barrier()

    def wb(y):
        for i in range(8):
            ch = t + NS * i

            @pl.when(ch < NZCH)
            def _():
                sl = pl.ds(ch * ZCH, ZCH)
                pltpu.sync_copy(acc.at[sl], y.at[sl])

    pl.when(c == 0)(lambda: wb(y0))
    pl.when(c == 1)(lambda: wb(y1))


_agg_call = pl.kernel(
    _agg_body,
    out_type=(jax.ShapeDtypeStruct((N, D), jnp.float32),
              jax.ShapeDtypeStruct((N, D), jnp.float32)),
    mesh=_mesh,
    scratch_types=[
        pltpu.VMEM_SHARED((N, D), jnp.float32),
        pltpu.VMEM((2, 2, WAVE, CH), jnp.int32),
        pltpu.VMEM((2, CH, D), jnp.float32),
        pltpu.SemaphoreType.DMA,
        pltpu.SemaphoreType.DMA,
    ],
)


def _nrm(d):
    return jnp.where(d > 0, lax.rsqrt(jnp.maximum(d, 1.0)), 0.0)


def _norm_body(x_ref, od0, id0, od1, id1, xs0, xs1, ns0, ns1, nd0, nd1):
    a = _nrm(od0[...])
    b = _nrm(od1[...])
    xv = x_ref[...]
    xs0[...] = xv * a
    xs1[...] = xv * b
    ns0[...] = a
    ns1[...] = b
    nd0[...] = _nrm(id0[...])
    nd1[...] = _nrm(id1[...])


_BLK = 1000
_vec_spec = pl.BlockSpec((_BLK, 1), lambda i: (i, 0))
_mat_spec = pl.BlockSpec((_BLK, D), lambda i: (i, 0))

_norm_call = pl.pallas_call(
    _norm_body,
    grid=(N // _BLK,),
    in_specs=[_mat_spec, _vec_spec, _vec_spec, _vec_spec, _vec_spec],
    out_specs=[_mat_spec, _mat_spec, _vec_spec, _vec_spec, _vec_spec, _vec_spec],
    out_shape=[jax.ShapeDtypeStruct((N, D), jnp.float32),
               jax.ShapeDtypeStruct((N, D), jnp.float32),
               jax.ShapeDtypeStruct((N, 1), jnp.float32),
               jax.ShapeDtypeStruct((N, 1), jnp.float32),
               jax.ShapeDtypeStruct((N, 1), jnp.float32),
               jax.ShapeDtypeStruct((N, 1), jnp.float32)],
)


def _w_spec():
    return pl.BlockSpec((D, D), lambda i: (0, 0))


def _comb0_body(y0, y1, nd0, nd1, ns0, ns1, w0, w1, bs, hs0, hs1):
    h = jnp.dot(y0[...] * nd0[...], w0[...], preferred_element_type=jnp.float32)
    h = h + jnp.dot(y1[...] * nd1[...], w1[...], preferred_element_type=jnp.float32)
    h = jnp.maximum(h + bs[...], 0.0)
    hs0[...] = h * ns0[...]
    hs1[...] = h * ns1[...]


_comb0_call = pl.pallas_call(
    _comb0_body,
    grid=(N // _BLK,),
    in_specs=[_mat_spec, _mat_spec, _vec_spec, _vec_spec, _vec_spec, _vec_spec,
              _w_spec(), _w_spec(), pl.BlockSpec((1, D), lambda i: (0, 0))],
    out_specs=[_mat_spec, _mat_spec],
    out_shape=[jax.ShapeDtypeStruct((N, D), jnp.float32),
               jax.ShapeDtypeStruct((N, D), jnp.float32)],
)


def _comb1_body(y0, y1, nd0, nd1, w0, w1, bs, out):
    h = jnp.dot(y0[...] * nd0[...], w0[...], preferred_element_type=jnp.float32)
    h = h + jnp.dot(y1[...] * nd1[...], w1[...], preferred_element_type=jnp.float32)
    out[...] = h + bs[...]


_comb1_call = pl.pallas_call(
    _comb1_body,
    grid=(N // _BLK,),
    in_specs=[_mat_spec, _mat_spec, _vec_spec, _vec_spec,
              _w_spec(), _w_spec(), pl.BlockSpec((1, D), lambda i: (0, 0))],
    out_specs=_mat_spec,
    out_shape=jax.ShapeDtypeStruct((N, D), jnp.float32),
)


@jax.jit
def kernel(x, edge_index_r0, edge_index_r1, W0_r0, b0_r0, W0_r1, b0_r1,
           W1_r0, b1_r0, W1_r1, b1_r1):
    e0 = edge_index_r0.reshape(2, CROWS, CH)
    e1 = edge_index_r1.reshape(2, CROWS, CH)
    z_n = jnp.zeros((NPAD,), jnp.float32)
    znd = jnp.zeros((N, D), jnp.float32)

    og0, ig0, og1, ig1 = _deg_call(e0, e1, z_n)
    od0 = og0[:N].reshape(N, 1)
    id0 = ig0[:N].reshape(N, 1)
    od1 = og1[:N].reshape(N, 1)
    id1 = ig1[:N].reshape(N, 1)

    xs0, xs1, ns0, ns1, nd0, nd1 = _norm_call(x, od0, id0, od1, id1)
    y00, y01 = _agg_call(xs0, xs1, e0, e1, znd)
    bs0 = (b0_r0 + b0_r1).reshape(1, D)
    hs0, hs1 = _comb0_call(y00, y01, nd0, nd1, ns0, ns1, W0_r0, W0_r1, bs0)
    y10, y11 = _agg_call(hs0, hs1, e0, e1, znd)
    bs1 = (b1_r0 + b1_r1).reshape(1, D)
    return _comb1_call(y10, y11, nd0, nd1, W1_r0, W1_r1, bs1)


# speedup vs baseline: 8.1815x; 1.0012x over previous
import jax
import jax.numpy as jnp
from jax import lax
from jax.experimental import pallas as pl
from jax.experimental.pallas import tpu as pltpu
from jax.experimental.pallas import tpu_sc as plsc

N = 10000
D = 128
E = 160000

NS = 16
CH = 125
CROWS = E // CH
CROW_PT = CROWS // NS
WAVE = 8
IT_PT = CROW_PT // WAVE
NPAD = NS * 640
ZCH = 80
NZCH = N // ZCH

_mesh = plsc.VectorSubcoreMesh(core_axis_name="c", subcore_axis_name="s")


def _deg_body(e0, e1, z_n, og0, ig0, og1, ig1, da, db, idx_v, ones_v, ssem):
    c = lax.axis_index("c")
    t = lax.axis_index("s")
    for j in range(128 // 16):
        ones_v[pl.ds(j * 16, 16)] = jnp.full((16,), 1.0, jnp.float32)
    own = pl.ds(t * 640, 640)
    pltpu.sync_copy(z_n.at[own], da.at[own])
    pltpu.sync_copy(z_n.at[own], db.at[own])
    plsc.subcore_barrier()
    ones_sl = ones_v.at[pl.ds(0, CH)]

    def run(e):
        @pl.loop(0, IT_PT)
        def body(i):
            rb = t * CROW_PT + i * WAVE
            pltpu.sync_copy(e.at[0, pl.ds(rb, WAVE)], idx_v.at[0])
            pltpu.sync_copy(e.at[1, pl.ds(rb, WAVE)], idx_v.at[1])
            hs = [pltpu.async_copy(ones_sl, da.at[idx_v.at[0, j]], ssem, add=True)
                  for j in range(WAVE)]
            hs += [pltpu.async_copy(ones_sl, db.at[idx_v.at[1, j]], ssem, add=True)
                   for j in range(WAVE)]
            for h in hs:
                h.wait()

    pl.when(c == 0)(lambda: run(e0))
    pl.when(c == 1)(lambda: run(e1))
    plsc.subcore_barrier()

    def wb(og, ig):
        pltpu.sync_copy(da.at[own], og.at[own])
        pltpu.sync_copy(db.at[own], ig.at[own])

    pl.when(c == 0)(lambda: wb(og0, ig0))
    pl.when(c == 1)(lambda: wb(og1, ig1))


_deg_call = pl.kernel(
    _deg_body,
    out_type=(jax.ShapeDtypeStruct((NPAD,), jnp.float32),
              jax.ShapeDtypeStruct((NPAD,), jnp.float32),
              jax.ShapeDtypeStruct((NPAD,), jnp.float32),
              jax.ShapeDtypeStruct((NPAD,), jnp.float32)),
    mesh=_mesh,
    scratch_types=[
        pltpu.VMEM_SHARED((NPAD,), jnp.float32),
        pltpu.VMEM_SHARED((NPAD,), jnp.float32),
        pltpu.VMEM((2, WAVE, CH), jnp.int32),
        pltpu.VMEM((128,), jnp.float32),
        pltpu.SemaphoreType.DMA,
    ],
)


def _maybe(pred, fn):
    def run():
        fn()

    if pred is None:
        run()
    else:
        pl.when(pred)(run)


def _agg_body(xs0, xs1, e0, e1, znd, y0, y1, acc, idx_v, rows_v, gsem, isem,
              ssem):
    c = lax.axis_index("c")
    t = lax.axis_index("s")
    for i in range(8):
        ch = t + NS * i

        @pl.when(ch < NZCH)
        def _():
            sl = pl.ds(ch * ZCH, ZCH)
            pltpu.sync_copy(znd.at[sl], acc.at[sl])

    plsc.subcore_barrier()
    base = t * CROW_PT

    def run(xs, e):
        pltpu.sync_copy(e.at[:, pl.ds(base, WAVE)], idx_v.at[0])
        pltpu.async_copy(xs.at[idx_v.at[0, 0, 0]], rows_v.at[0], gsem)

        def wave(w, pw, pre_pred, next_pred, first_pred):
            po = 1 - pw
            for q in range(WAVE):
                pltpu.make_async_copy(xs.at[idx_v.at[pw, 0, q]],
                                      rows_v.at[q % 2], gsem).wait()
                _maybe(first_pred if q == 0 else None,
                       lambda: pltpu.make_async_copy(
                           rows_v.at[(q + 1) % 2],
                           acc.at[idx_v.at[pw, 1, q]], ssem).wait())
                if q == 0:
                    _maybe(pre_pred, lambda: pltpu.async_copy(
                        e.at[:, pl.ds(base + (w + 1) * WAVE, WAVE)],
                        idx_v.at[po], isem))
                if q < WAVE - 1:
                    pltpu.async_copy(xs.at[idx_v.at[pw, 0, q + 1]],
                                     rows_v.at[(q + 1) % 2], gsem)
                if q == WAVE - 2:
                    _maybe(pre_pred, lambda: pltpu.make_async_copy(
                        e.at[:, pl.ds(base, WAVE)], idx_v.at[po], isem).wait())
                if q == WAVE - 1:
                    _maybe(next_pred, lambda: pltpu.async_copy(
                        xs.at[idx_v.at[po, 0, 0]], rows_v.at[0], gsem))
                pltpu.async_copy(rows_v.at[q % 2], acc.at[idx_v.at[pw, 1, q]],
                                 ssem, add=True)

        @pl.loop(0, IT_PT // 2)
        def pair(i):
            not_last = i < IT_PT // 2 - 1
            wave(2 * i, 0, None, None, i > 0)
            wave(2 * i + 1, 1, not_last, not_last, None)

        pltpu.make_async_copy(rows_v.at[1], acc.at[idx_v.at[1, 1, WAVE - 1]],
                              ssem).wait()

    pl.when(c == 0)(lambda: run(xs0, e0))
    pl.when(c == 1)(lambda: run(xs1, e1))
    plsc.subcore_barrier()

    def wb(y):
        for i in range(8):
            ch = t + NS * i

            @pl.when(ch < NZCH)
            def _():
                sl = pl.ds(ch * ZCH, ZCH)
                pltpu.sync_copy(acc.at[sl], y.at[sl])

    pl.when(c == 0)(lambda: wb(y0))
    pl.when(c == 1)(lambda: wb(y1))


_agg_call = pl.kernel(
    _agg_body,
    out_type=(jax.ShapeDtypeStruct((N, D), jnp.float32),
              jax.ShapeDtypeStruct((N, D), jnp.float32)),
    mesh=_mesh,
    scratch_types=[
        pltpu.VMEM_SHARED((N, D), jnp.float32),
        pltpu.VMEM((2, 2, WAVE, CH), jnp.int32),
        pltpu.VMEM((2, CH, D), jnp.float32),
        pltpu.SemaphoreType.DMA,
        pltpu.SemaphoreType.DMA,
        pltpu.SemaphoreType.DMA,
    ],
)


def _nrm(d):
    return jnp.where(d > 0, lax.rsqrt(jnp.maximum(d, 1.0)), 0.0)


def _norm_body(x_ref, od0, id0, od1, id1, xs0, xs1, ns0, ns1, nd0, nd1):
    a = _nrm(od0[...])
    b = _nrm(od1[...])
    xv = x_ref[...]
    xs0[...] = xv * a
    xs1[...] = xv * b
    ns0[...] = a
    ns1[...] = b
    nd0[...] = _nrm(id0[...])
    nd1[...] = _nrm(id1[...])


_BLK = 1000
_vec_spec = pl.BlockSpec((_BLK, 1), lambda i: (i, 0))
_mat_spec = pl.BlockSpec((_BLK, D), lambda i: (i, 0))

_norm_call = pl.pallas_call(
    _norm_body,
    grid=(N // _BLK,),
    in_specs=[_mat_spec, _vec_spec, _vec_spec, _vec_spec, _vec_spec],
    out_specs=[_mat_spec, _mat_spec, _vec_spec, _vec_spec, _vec_spec, _vec_spec],
    out_shape=[jax.ShapeDtypeStruct((N, D), jnp.float32),
               jax.ShapeDtypeStruct((N, D), jnp.float32),
               jax.ShapeDtypeStruct((N, 1), jnp.float32),
               jax.ShapeDtypeStruct((N, 1), jnp.float32),
               jax.ShapeDtypeStruct((N, 1), jnp.float32),
               jax.ShapeDtypeStruct((N, 1), jnp.float32)],
)


def _w_spec():
    return pl.BlockSpec((D, D), lambda i: (0, 0))


def _comb0_body(y0, y1, nd0, nd1, ns0, ns1, w0, w1, bs, hs0, hs1):
    h = jnp.dot(y0[...] * nd0[...], w0[...], preferred_element_type=jnp.float32)
    h = h + jnp.dot(y1[...] * nd1[...], w1[...], preferred_element_type=jnp.float32)
    h = jnp.maximum(h + bs[...], 0.0)
    hs0[...] = h * ns0[...]
    hs1[...] = h * ns1[...]


_comb0_call = pl.pallas_call(
    _comb0_body,
    grid=(N // _BLK,),
    in_specs=[_mat_spec, _mat_spec, _vec_spec, _vec_spec, _vec_spec, _vec_spec,
              _w_spec(), _w_spec(), pl.BlockSpec((1, D), lambda i: (0, 0))],
    out_specs=[_mat_spec, _mat_spec],
    out_shape=[jax.ShapeDtypeStruct((N, D), jnp.float32),
               jax.ShapeDtypeStruct((N, D), jnp.float32)],
)


def _comb1_body(y0, y1, nd0, nd1, w0, w1, bs, out):
    h = jnp.dot(y0[...] * nd0[...], w0[...], preferred_element_type=jnp.float32)
    h = h + jnp.dot(y1[...] * nd1[...], w1[...], preferred_element_type=jnp.float32)
    out[...] = h + bs[...]


_comb1_call = pl.pallas_call(
    _comb1_body,
    grid=(N // _BLK,),
    in_specs=[_mat_spec, _mat_spec, _vec_spec, _vec_spec,
              _w_spec(), _w_spec(), pl.BlockSpec((1, D), lambda i: (0, 0))],
    out_specs=_mat_spec,
    out_shape=jax.ShapeDtypeStruct((N, D), jnp.float32),
)


@jax.jit
def kernel(x, edge_index_r0, edge_index_r1, W0_r0, b0_r0, W0_r1, b0_r1,
           W1_r0, b1_r0, W1_r1, b1_r1):
    e0 = edge_index_r0.reshape(2, CROWS, CH)
    e1 = edge_index_r1.reshape(2, CROWS, CH)
    z_n = jnp.zeros((NPAD,), jnp.float32)
    znd = jnp.zeros((N, D), jnp.float32)

    og0, ig0, og1, ig1 = _deg_call(e0, e1, z_n)
    od0 = og0[:N].reshape(N, 1)
    id0 = ig0[:N].reshape(N, 1)
    od1 = og1[:N].reshape(N, 1)
    id1 = ig1[:N].reshape(N, 1)

    xs0, xs1, ns0, ns1, nd0, nd1 = _norm_call(x, od0, id0, od1, id1)
    y00, y01 = _agg_call(xs0, xs1, e0, e1, znd)
    bs0 = (b0_r0 + b0_r1).reshape(1, D)
    hs0, hs1 = _comb0_call(y00, y01, nd0, nd1, ns0, ns1, W0_r0, W0_r1, bs0)
    y10, y11 = _agg_call(hs0, hs1, e0, e1, znd)
    bs1 = (b1_r0 + b1_r1).reshape(1, D)
    return _comb1_call(y10, y11, nd0, nd1, W1_r0, W1_r1, bs1)


# speedup vs baseline: 8.5717x; 1.0477x over previous
import jax
import jax.numpy as jnp
from jax import lax
from jax.experimental import pallas as pl
from jax.experimental.pallas import tpu as pltpu
from jax.experimental.pallas import tpu_sc as plsc

N = 10000
D = 128
E = 160000

NS = 16
CH = 125
CROWS = E // CH
CROW_PT = CROWS // NS
WAVE = 8
IT_PT = CROW_PT // WAVE
NPAD = NS * 640
ZCH = 80
NZCH = N // ZCH

_mesh = plsc.VectorSubcoreMesh(core_axis_name="c", subcore_axis_name="s")


def _maybe(pred, fn):
    def run():
        fn()

    if pred is None:
        run()
    else:
        pl.when(pred)(run)


def _deg_body(e0, e1, z_n, og0, ig0, og1, ig1, da, db, idx_v, ones_v, ssem,
              isem):
    c = lax.axis_index("c")
    t = lax.axis_index("s")
    for j in range(128 // 16):
        ones_v[pl.ds(j * 16, 16)] = jnp.full((16,), 1.0, jnp.float32)
    own = pl.ds(t * 640, 640)
    pltpu.sync_copy(z_n.at[own], da.at[own])
    pltpu.sync_copy(z_n.at[own], db.at[own])
    plsc.subcore_barrier()
    ones_sl = ones_v.at[pl.ds(0, CH)]
    base0 = t * CROW_PT

    def run(e):
        pltpu.sync_copy(e.at[:, pl.ds(base0, WAVE)], idx_v.at[0])

        def wave(w, pw, drain_pred, iwait_pred, pre_pred):
            po = 1 - pw
            _maybe(drain_pred, lambda: [
                pltpu.make_async_copy(ones_sl, da.at[idx_v.at[po, 0, j]],
                                      ssem).wait()
                for j in range(2 * WAVE)])
            _maybe(pre_pred, lambda: pltpu.async_copy(
                e.at[:, pl.ds(base0 + (w + 1) * WAVE, WAVE)],
                idx_v.at[po], isem))
            _maybe(iwait_pred, lambda: pltpu.make_async_copy(
                e.at[:, pl.ds(base0, WAVE)], idx_v.at[pw], isem).wait())
            for j in range(WAVE):
                pltpu.async_copy(ones_sl, da.at[idx_v.at[pw, 0, j]], ssem,
                                 add=True)
                pltpu.async_copy(ones_sl, db.at[idx_v.at[pw, 1, j]], ssem,
                                 add=True)

        @pl.loop(0, IT_PT // 2)
        def pair(i):
            not_last = i < IT_PT // 2 - 1
            wave(2 * i, 0, i > 0, i > 0, None)
            wave(2 * i + 1, 1, None, None, not_last)

        for j in range(2 * WAVE):
            pltpu.make_async_copy(ones_sl, da.at[idx_v.at[1, 0, j % WAVE]],
                                  ssem).wait()

    pl.when(c == 0)(lambda: run(e0))
    pl.when(c == 1)(lambda: run(e1))
    plsc.subcore_barrier()

    def wb(og, ig):
        pltpu.sync_copy(da.at[own], og.at[own])
        pltpu.sync_copy(db.at[own], ig.at[own])

    pl.when(c == 0)(lambda: wb(og0, ig0))
    pl.when(c == 1)(lambda: wb(og1, ig1))


_deg_call = pl.kernel(
    _deg_body,
    out_type=(jax.ShapeDtypeStruct((NPAD,), jnp.float32),
              jax.ShapeDtypeStruct((NPAD,), jnp.float32),
              jax.ShapeDtypeStruct((NPAD,), jnp.float32),
              jax.ShapeDtypeStruct((NPAD,), jnp.float32)),
    mesh=_mesh,
    scratch_types=[
        pltpu.VMEM_SHARED((NPAD,), jnp.float32),
        pltpu.VMEM_SHARED((NPAD,), jnp.float32),
        pltpu.VMEM((2, 2, WAVE, CH), jnp.int32),
        pltpu.VMEM((128,), jnp.float32),
        pltpu.SemaphoreType.DMA,
        pltpu.SemaphoreType.DMA,
    ],
)


def _agg_body(xs0, xs1, e0, e1, znd, y0, y1, acc, idx_v, rows_v, gsem, isem,
              ssem):
    c = lax.axis_index("c")
    t = lax.axis_index("s")
    for i in range(8):
        ch = t + NS * i

        @pl.when(ch < NZCH)
        def _():
            sl = pl.ds(ch * ZCH, ZCH)
            pltpu.sync_copy(znd.at[sl], acc.at[sl])

    plsc.subcore_barrier()
    base = t * CROW_PT

    def run(xs, e):
        pltpu.sync_copy(e.at[:, pl.ds(base, WAVE)], idx_v.at[0])
        pltpu.async_copy(xs.at[idx_v.at[0, 0, 0]], rows_v.at[0], gsem)

        def wave(w, pw, pre_pred, next_pred, first_pred):
            po = 1 - pw
            for q in range(WAVE):
                pltpu.make_async_copy(xs.at[idx_v.at[pw, 0, q]],
                                      rows_v.at[q % 2], gsem).wait()
                _maybe(first_pred if q == 0 else None,
                       lambda: pltpu.make_async_copy(
                           rows_v.at[(q + 1) % 2],
                           acc.at[idx_v.at[pw, 1, q]], ssem).wait())
                if q == 0:
                    _maybe(pre_pred, lambda: pltpu.async_copy(
                        e.at[:, pl.ds(base + (w + 1) * WAVE, WAVE)],
                        idx_v.at[po], isem))
                if q < WAVE - 1:
                    pltpu.async_copy(xs.at[idx_v.at[pw, 0, q + 1]],
                                     rows_v.at[(q + 1) % 2], gsem)
                if q == WAVE - 2:
                    _maybe(pre_pred, lambda: pltpu.make_async_copy(
                        e.at[:, pl.ds(base, WAVE)], idx_v.at[po], isem).wait())
                if q == WAVE - 1:
                    _maybe(next_pred, lambda: pltpu.async_copy(
                        xs.at[idx_v.at[po, 0, 0]], rows_v.at[0], gsem))
                pltpu.async_copy(rows_v.at[q % 2], acc.at[idx_v.at[pw, 1, q]],
                                 ssem, add=True)

        @pl.loop(0, IT_PT // 2)
        def pair(i):
            not_last = i < IT_PT // 2 - 1
            wave(2 * i, 0, None, None, i > 0)
            wave(2 * i + 1, 1, not_last, not_last, None)

        pltpu.make_async_copy(rows_v.at[1], acc.at[idx_v.at[1, 1, WAVE - 1]],
                              ssem).wait()

    pl.when(c == 0)(lambda: run(xs0, e0))
    pl.when(c == 1)(lambda: run(xs1, e1))
    plsc.subcore_barrier()

    def wb(y):
        for i in range(8):
            ch = t + NS * i

            @pl.when(ch < NZCH)
            def _():
                sl = pl.ds(ch * ZCH, ZCH)
                pltpu.sync_copy(acc.at[sl], y.at[sl])

    pl.when(c == 0)(lambda: wb(y0))
    pl.when(c == 1)(lambda: wb(y1))


_agg_call = pl.kernel(
    _agg_body,
    out_type=(jax.ShapeDtypeStruct((N, D), jnp.float32),
              jax.ShapeDtypeStruct((N, D), jnp.float32)),
    mesh=_mesh,
    scratch_types=[
        pltpu.VMEM_SHARED((N, D), jnp.float32),
        pltpu.VMEM((2, 2, WAVE, CH), jnp.int32),
        pltpu.VMEM((2, CH, D), jnp.float32),
        pltpu.SemaphoreType.DMA,
        pltpu.SemaphoreType.DMA,
        pltpu.SemaphoreType.DMA,
    ],
)


def _nrm(d):
    return jnp.where(d > 0, lax.rsqrt(jnp.maximum(d, 1.0)), 0.0)


def _norm_body(x_ref, od0, id0, od1, id1, xs0, xs1, ns0, ns1, nd0, nd1):
    a = _nrm(od0[...])
    b = _nrm(od1[...])
    xv = x_ref[...]
    xs0[...] = xv * a
    xs1[...] = xv * b
    ns0[...] = a
    ns1[...] = b
    nd0[...] = _nrm(id0[...])
    nd1[...] = _nrm(id1[...])


_BLK = 1000
_vec_spec = pl.BlockSpec((_BLK, 1), lambda i: (i, 0))
_mat_spec = pl.BlockSpec((_BLK, D), lambda i: (i, 0))

_norm_call = pl.pallas_call(
    _norm_body,
    grid=(N // _BLK,),
    in_specs=[_mat_spec, _vec_spec, _vec_spec, _vec_spec, _vec_spec],
    out_specs=[_mat_spec, _mat_spec, _vec_spec, _vec_spec, _vec_spec, _vec_spec],
    out_shape=[jax.ShapeDtypeStruct((N, D), jnp.float32),
               jax.ShapeDtypeStruct((N, D), jnp.float32),
               jax.ShapeDtypeStruct((N, 1), jnp.float32),
               jax.ShapeDtypeStruct((N, 1), jnp.float32),
               jax.ShapeDtypeStruct((N, 1), jnp.float32),
               jax.ShapeDtypeStruct((N, 1), jnp.float32)],
)


def _w_spec():
    return pl.BlockSpec((D, D), lambda i: (0, 0))


def _comb0_body(y0, y1, nd0, nd1, ns0, ns1, w0, w1, bs, hs0, hs1):
    h = jnp.dot(y0[...] * nd0[...], w0[...], preferred_element_type=jnp.float32)
    h = h + jnp.dot(y1[...] * nd1[...], w1[...], preferred_element_type=jnp.float32)
    h = jnp.maximum(h + bs[...], 0.0)
    hs0[...] = h * ns0[...]
    hs1[...] = h * ns1[...]


_comb0_call = pl.pallas_call(
    _comb0_body,
    grid=(N // _BLK,),
    in_specs=[_mat_spec, _mat_spec, _vec_spec, _vec_spec, _vec_spec, _vec_spec,
              _w_spec(), _w_spec(), pl.BlockSpec((1, D), lambda i: (0, 0))],
    out_specs=[_mat_spec, _mat_spec],
    out_shape=[jax.ShapeDtypeStruct((N, D), jnp.float32),
               jax.ShapeDtypeStruct((N, D), jnp.float32)],
)


def _comb1_body(y0, y1, nd0, nd1, w0, w1, bs, out):
    h = jnp.dot(y0[...] * nd0[...], w0[...], preferred_element_type=jnp.float32)
    h = h + jnp.dot(y1[...] * nd1[...], w1[...], preferred_element_type=jnp.float32)
    out[...] = h + bs[...]


_comb1_call = pl.pallas_call(
    _comb1_body,
    grid=(N // _BLK,),
    in_specs=[_mat_spec, _mat_spec, _vec_spec, _vec_spec,
              _w_spec(), _w_spec(), pl.BlockSpec((1, D), lambda i: (0, 0))],
    out_specs=_mat_spec,
    out_shape=jax.ShapeDtypeStruct((N, D), jnp.float32),
)


@jax.jit
def kernel(x, edge_index_r0, edge_index_r1, W0_r0, b0_r0, W0_r1, b0_r1,
           W1_r0, b1_r0, W1_r1, b1_r1):
    e0 = edge_index_r0.reshape(2, CROWS, CH)
    e1 = edge_index_r1.reshape(2, CROWS, CH)
    z_n = jnp.zeros((NPAD,), jnp.float32)
    znd = jnp.zeros((N, D), jnp.float32)

    og0, ig0, og1, ig1 = _deg_call(e0, e1, z_n)
    od0 = og0.reshape(NPAD, 1)
    id0 = ig0.reshape(NPAD, 1)
    od1 = og1.reshape(NPAD, 1)
    id1 = ig1.reshape(NPAD, 1)

    xs0, xs1, ns0, ns1, nd0, nd1 = _norm_call(x, od0, id0, od1, id1)
    y00, y01 = _agg_call(xs0, xs1, e0, e1, znd)
    bs0 = (b0_r0 + b0_r1).reshape(1, D)
    hs0, hs1 = _comb0_call(y00, y01, nd0, nd1, ns0, ns1, W0_r0, W0_r1, bs0)
    y10, y11 = _agg_call(hs0, hs1, e0, e1, znd)
    bs1 = (b1_r0 + b1_r1).reshape(1, D)
    return _comb1_call(y10, y11, nd0, nd1, W1_r0, W1_r1, bs1)


# speedup vs baseline: 8.7015x; 1.0151x over previous
import jax
import jax.numpy as jnp
from jax import lax
from jax.experimental import pallas as pl
from jax.experimental.pallas import tpu as pltpu
from jax.experimental.pallas import tpu_sc as plsc

N = 10000
D = 128
E = 160000

NS = 16
CH = 125
CROWS = E // CH
CROW_PT = CROWS // NS
WAVE = 8
IT_PT = CROW_PT // WAVE
NPAD = NS * 640
ZCH = 80
NZCH = N // ZCH

_mesh = plsc.VectorSubcoreMesh(core_axis_name="c", subcore_axis_name="s")


def _maybe(pred, fn):
    def run():
        fn()

    if pred is None:
        run()
    else:
        pl.when(pred)(run)


def _deg_body(e0, e1, z_n, og0, ig0, og1, ig1, da, db, idx_v, ones_v, ssem,
              isem):
    c = lax.axis_index("c")
    t = lax.axis_index("s")
    for j in range(128 // 16):
        ones_v[pl.ds(j * 16, 16)] = jnp.full((16,), 1.0, jnp.float32)
    own = pl.ds(t * 640, 640)
    pltpu.sync_copy(z_n.at[own], da.at[own])
    pltpu.sync_copy(z_n.at[own], db.at[own])
    plsc.subcore_barrier()
    ones_sl = ones_v.at[pl.ds(0, CH)]
    base0 = t * CROW_PT

    def run(e):
        pltpu.sync_copy(e.at[:, pl.ds(base0, WAVE)], idx_v.at[0])

        def wave(w, pw, drain_pred, iwait_pred, pre_pred):
            po = 1 - pw
            _maybe(drain_pred, lambda: [
                pltpu.make_async_copy(ones_sl, da.at[idx_v.at[po, 0, j]],
                                      ssem).wait()
                for j in range(2 * WAVE)])
            _maybe(pre_pred, lambda: pltpu.async_copy(
                e.at[:, pl.ds(base0 + (w + 1) * WAVE, WAVE)],
                idx_v.at[po], isem))
            _maybe(iwait_pred, lambda: pltpu.make_async_copy(
                e.at[:, pl.ds(base0, WAVE)], idx_v.at[pw], isem).wait())
            for j in range(WAVE):
                pltpu.async_copy(ones_sl, da.at[idx_v.at[pw, 0, j]], ssem,
                                 add=True)
                pltpu.async_copy(ones_sl, db.at[idx_v.at[pw, 1, j]], ssem,
                                 add=True)

        @pl.loop(0, IT_PT // 2)
        def pair(i):
            not_last = i < IT_PT // 2 - 1
            wave(2 * i, 0, i > 0, i > 0, None)
            wave(2 * i + 1, 1, None, None, not_last)

        for j in range(2 * WAVE):
            pltpu.make_async_copy(ones_sl, da.at[idx_v.at[1, 0, j % WAVE]],
                                  ssem).wait()

    pl.when(c == 0)(lambda: run(e0))
    pl.when(c == 1)(lambda: run(e1))
    plsc.subcore_barrier()

    def wb(og, ig):
        pltpu.sync_copy(da.at[own], og.at[own])
        pltpu.sync_copy(db.at[own], ig.at[own])

    pl.when(c == 0)(lambda: wb(og0, ig0))
    pl.when(c == 1)(lambda: wb(og1, ig1))


_deg_call = pl.kernel(
    _deg_body,
    out_type=(jax.ShapeDtypeStruct((NPAD,), jnp.float32),
              jax.ShapeDtypeStruct((NPAD,), jnp.float32),
              jax.ShapeDtypeStruct((NPAD,), jnp.float32),
              jax.ShapeDtypeStruct((NPAD,), jnp.float32)),
    mesh=_mesh,
    scratch_types=[
        pltpu.VMEM_SHARED((NPAD,), jnp.float32),
        pltpu.VMEM_SHARED((NPAD,), jnp.float32),
        pltpu.VMEM((2, 2, WAVE, CH), jnp.int32),
        pltpu.VMEM((128,), jnp.float32),
        pltpu.SemaphoreType.DMA,
        pltpu.SemaphoreType.DMA,
    ],
)


def _agg_body(xs0, xs1, e0, e1, znd, y0, y1, acc, idx_v, rows_v, gsem, isem,
              ssem):
    c = lax.axis_index("c")
    t = lax.axis_index("s")
    for i in range(8):
        ch = t + NS * i

        @pl.when(ch < NZCH)
        def _():
            sl = pl.ds(ch * ZCH, ZCH)
            pltpu.sync_copy(znd.at[sl], acc.at[sl])

    plsc.subcore_barrier()
    base = t * CROW_PT

    def run(xs, e):
        pltpu.sync_copy(e.at[:, pl.ds(base, WAVE)], idx_v.at[0])
        pltpu.async_copy(xs.at[idx_v.at[0, 0, 0]], rows_v.at[0], gsem)

        def wave(w, pw, pre_pred, next_pred, first_pred):
            po = 1 - pw
            for q in range(WAVE):
                pltpu.make_async_copy(xs.at[idx_v.at[pw, 0, q]],
                                      rows_v.at[q % 2], gsem).wait()
                _maybe(first_pred if q == 0 else None,
                       lambda: pltpu.make_async_copy(
                           rows_v.at[(q + 1) % 2],
                           acc.at[idx_v.at[pw, 1, q]], ssem).wait())
                if q == 0:
                    _maybe(pre_pred, lambda: pltpu.async_copy(
                        e.at[:, pl.ds(base + (w + 1) * WAVE, WAVE)],
                        idx_v.at[po], isem))
                if q < WAVE - 1:
                    pltpu.async_copy(xs.at[idx_v.at[pw, 0, q + 1]],
                                     rows_v.at[(q + 1) % 2], gsem)
                if q == WAVE - 2:
                    _maybe(pre_pred, lambda: pltpu.make_async_copy(
                        e.at[:, pl.ds(base, WAVE)], idx_v.at[po], isem).wait())
                if q == WAVE - 1:
                    _maybe(next_pred, lambda: pltpu.async_copy(
                        xs.at[idx_v.at[po, 0, 0]], rows_v.at[0], gsem))
                pltpu.async_copy(rows_v.at[q % 2], acc.at[idx_v.at[pw, 1, q]],
                                 ssem, add=True)

        @pl.loop(0, IT_PT // 2)
        def pair(i):
            not_last = i < IT_PT // 2 - 1
            wave(2 * i, 0, None, None, i > 0)
            wave(2 * i + 1, 1, not_last, not_last, None)

        pltpu.make_async_copy(rows_v.at[1], acc.at[idx_v.at[1, 1, WAVE - 1]],
                              ssem).wait()

    pl.when(c == 0)(lambda: run(xs0, e0))
    pl.when(c == 1)(lambda: run(xs1, e1))
    plsc.subcore_barrier()

    def wb(y):
        for i in range(8):
            ch = t + NS * i

            @pl.when(ch < NZCH)
            def _():
                sl = pl.ds(ch * ZCH, ZCH)
                pltpu.sync_copy(acc.at[sl], y.at[sl])

    pl.when(c == 0)(lambda: wb(y0))
    pl.when(c == 1)(lambda: wb(y1))


_agg_call = pl.kernel(
    _agg_body,
    out_type=(jax.ShapeDtypeStruct((N, D), jnp.float32),
              jax.ShapeDtypeStruct((N, D), jnp.float32)),
    mesh=_mesh,
    scratch_types=[
        pltpu.VMEM_SHARED((N, D), jnp.float32),
        pltpu.VMEM((2, 2, WAVE, CH), jnp.int32),
        pltpu.VMEM((2, CH, D), jnp.float32),
        pltpu.SemaphoreType.DMA,
        pltpu.SemaphoreType.DMA,
        pltpu.SemaphoreType.DMA,
    ],
)


def _nrm(d):
    return jnp.where(d > 0, lax.rsqrt(jnp.maximum(d, 1.0)), 0.0)


def _norm_body(x_ref, od0, id0, od1, id1, xs0, xs1, ns0, ns1, nd0, nd1):
    a = _nrm(od0[...])
    b = _nrm(od1[...])
    xv = x_ref[...]
    xs0[...] = xv * a
    xs1[...] = xv * b
    ns0[...] = a
    ns1[...] = b
    nd0[...] = _nrm(id0[...])
    nd1[...] = _nrm(id1[...])


_BLK = 2000
_vec_spec = pl.BlockSpec((_BLK, 1), lambda i: (i, 0))
_mat_spec = pl.BlockSpec((_BLK, D), lambda i: (i, 0))

_norm_call = pl.pallas_call(
    _norm_body,
    grid=(N // _BLK,),
    in_specs=[_mat_spec, _vec_spec, _vec_spec, _vec_spec, _vec_spec],
    out_specs=[_mat_spec, _mat_spec, _vec_spec, _vec_spec, _vec_spec, _vec_spec],
    out_shape=[jax.ShapeDtypeStruct((N, D), jnp.float32),
               jax.ShapeDtypeStruct((N, D), jnp.float32),
               jax.ShapeDtypeStruct((N, 1), jnp.float32),
               jax.ShapeDtypeStruct((N, 1), jnp.float32),
               jax.ShapeDtypeStruct((N, 1), jnp.float32),
               jax.ShapeDtypeStruct((N, 1), jnp.float32)],
)


def _w_spec():
    return pl.BlockSpec((D, D), lambda i: (0, 0))


def _comb0_body(y0, y1, nd0, nd1, ns0, ns1, w0, w1, bs, hs0, hs1):
    h = jnp.dot(y0[...] * nd0[...], w0[...], preferred_element_type=jnp.float32)
    h = h + jnp.dot(y1[...] * nd1[...], w1[...], preferred_element_type=jnp.float32)
    h = jnp.maximum(h + bs[...], 0.0)
    hs0[...] = h * ns0[...]
    hs1[...] = h * ns1[...]


_comb0_call = pl.pallas_call(
    _comb0_body,
    grid=(N // _BLK,),
    in_specs=[_mat_spec, _mat_spec, _vec_spec, _vec_spec, _vec_spec, _vec_spec,
              _w_spec(), _w_spec(), pl.BlockSpec((1, D), lambda i: (0, 0))],
    out_specs=[_mat_spec, _mat_spec],
    out_shape=[jax.ShapeDtypeStruct((N, D), jnp.float32),
               jax.ShapeDtypeStruct((N, D), jnp.float32)],
)


def _comb1_body(y0, y1, nd0, nd1, w0, w1, bs, out):
    h = jnp.dot(y0[...] * nd0[...], w0[...], preferred_element_type=jnp.float32)
    h = h + jnp.dot(y1[...] * nd1[...], w1[...], preferred_element_type=jnp.float32)
    out[...] = h + bs[...]


_comb1_call = pl.pallas_call(
    _comb1_body,
    grid=(N // _BLK,),
    in_specs=[_mat_spec, _mat_spec, _vec_spec, _vec_spec,
              _w_spec(), _w_spec(), pl.BlockSpec((1, D), lambda i: (0, 0))],
    out_specs=_mat_spec,
    out_shape=jax.ShapeDtypeStruct((N, D), jnp.float32),
)


@jax.jit
def kernel(x, edge_index_r0, edge_index_r1, W0_r0, b0_r0, W0_r1, b0_r1,
           W1_r0, b1_r0, W1_r1, b1_r1):
    e0 = edge_index_r0.reshape(2, CROWS, CH)
    e1 = edge_index_r1.reshape(2, CROWS, CH)
    z_n = jnp.zeros((NPAD,), jnp.float32)
    znd = jnp.zeros((N, D), jnp.float32)

    og0, ig0, og1, ig1 = _deg_call(e0, e1, z_n)
    od0 = og0.reshape(NPAD, 1)
    id0 = ig0.reshape(NPAD, 1)
    od1 = og1.reshape(NPAD, 1)
    id1 = ig1.reshape(NPAD, 1)

    xs0, xs1, ns0, ns1, nd0, nd1 = _norm_call(x, od0, id0, od1, id1)
    y00, y01 = _agg_call(xs0, xs1, e0, e1, znd)
    bs0 = (b0_r0 + b0_r1).reshape(1, D)
    hs0, hs1 = _comb0_call(y00, y01, nd0, nd1, ns0, ns1, W0_r0, W0_r1, bs0)
    y10, y11 = _agg_call(hs0, hs1, e0, e1, znd)
    bs1 = (b1_r0 + b1_r1).reshape(1, D)
    return _comb1_call(y10, y11, nd0, nd1, W1_r0, W1_r1, bs1)


# speedup vs baseline: 8.9450x; 1.0280x over previous
import jax
import jax.numpy as jnp
from jax import lax
from jax.experimental import pallas as pl
from jax.experimental.pallas import tpu as pltpu
from jax.experimental.pallas import tpu_sc as plsc

N = 10000
D = 128
E = 160000

NS = 16
CH = 125
CROWS = E // CH
CROW_PT = CROWS // NS
WAVE = 8
IT_PT = CROW_PT // WAVE
NPAD = NS * 640
ZCH = 80
NZCH = N // ZCH

_mesh = plsc.VectorSubcoreMesh(core_axis_name="c", subcore_axis_name="s")


def _maybe(pred, fn):
    def run():
        fn()

    if pred is None:
        run()
    else:
        pl.when(pred)(run)


def _deg_body(e0, e1, z_n, og0, ig0, og1, ig1, da, db, idx_v, ones_v, ssem,
              isem):
    c = lax.axis_index("c")
    t = lax.axis_index("s")
    for j in range(128 // 16):
        ones_v[pl.ds(j * 16, 16)] = jnp.full((16,), 1.0, jnp.float32)
    own = pl.ds(t * 640, 640)
    pltpu.sync_copy(z_n.at[own], da.at[own])
    pltpu.sync_copy(z_n.at[own], db.at[own])
    plsc.subcore_barrier()
    ones_sl = ones_v.at[pl.ds(0, CH)]
    base0 = t * CROW_PT

    def run(e):
        pltpu.sync_copy(e.at[:, pl.ds(base0, WAVE)], idx_v.at[0])

        def wave(w, pw, drain_pred, iwait_pred, pre_pred):
            po = 1 - pw
            _maybe(drain_pred, lambda: [
                pltpu.make_async_copy(ones_sl, da.at[idx_v.at[po, 0, j]],
                                      ssem).wait()
                for j in range(2 * WAVE)])
            _maybe(pre_pred, lambda: pltpu.async_copy(
                e.at[:, pl.ds(base0 + (w + 1) * WAVE, WAVE)],
                idx_v.at[po], isem))
            _maybe(iwait_pred, lambda: pltpu.make_async_copy(
                e.at[:, pl.ds(base0, WAVE)], idx_v.at[pw], isem).wait())
            for j in range(WAVE):
                pltpu.async_copy(ones_sl, da.at[idx_v.at[pw, 0, j]], ssem,
                                 add=True)
                pltpu.async_copy(ones_sl, db.at[idx_v.at[pw, 1, j]], ssem,
                                 add=True)

        @pl.loop(0, IT_PT // 2)
        def pair(i):
            not_last = i < IT_PT // 2 - 1
            wave(2 * i, 0, i > 0, i > 0, None)
            wave(2 * i + 1, 1, None, None, not_last)

        for j in range(2 * WAVE):
            pltpu.make_async_copy(ones_sl, da.at[idx_v.at[1, 0, j % WAVE]],
                                  ssem).wait()

    pl.when(c == 0)(lambda: run(e0))
    pl.when(c == 1)(lambda: run(e1))
    plsc.subcore_barrier()

    def wb(og, ig):
        pltpu.sync_copy(da.at[own], og.at[own])
        pltpu.sync_copy(db.at[own], ig.at[own])

    pl.when(c == 0)(lambda: wb(og0, ig0))
    pl.when(c == 1)(lambda: wb(og1, ig1))


_deg_call = pl.kernel(
    _deg_body,
    out_type=(jax.ShapeDtypeStruct((NPAD,), jnp.float32),
              jax.ShapeDtypeStruct((NPAD,), jnp.float32),
              jax.ShapeDtypeStruct((NPAD,), jnp.float32),
              jax.ShapeDtypeStruct((NPAD,), jnp.float32)),
    mesh=_mesh,
    scratch_types=[
        pltpu.VMEM_SHARED((NPAD,), jnp.float32),
        pltpu.VMEM_SHARED((NPAD,), jnp.float32),
        pltpu.VMEM((2, 2, WAVE, CH), jnp.int32),
        pltpu.VMEM((128,), jnp.float32),
        pltpu.SemaphoreType.DMA,
        pltpu.SemaphoreType.DMA,
    ],
)


def _agg_body(xs0, xs1, e0, e1, znd, y0, y1, acc, idx_v, rows_v, gsem, isem,
              ssem):
    c = lax.axis_index("c")
    t = lax.axis_index("s")
    for i in range(8):
        ch = t + NS * i
        _maybe(ch < NZCH,
               lambda sl=pl.ds(ch * ZCH, ZCH): pltpu.async_copy(
                   znd.at[sl], acc.at[sl], isem))
    base = t * CROW_PT

    def run(xs, e):
        pltpu.sync_copy(e.at[:, pl.ds(base, WAVE)], idx_v.at[0])
        pltpu.async_copy(xs.at[idx_v.at[0, 0, 0]], rows_v.at[0], gsem)
        for i in range(8):
            ch = t + NS * i
            _maybe(ch < NZCH,
                   lambda sl=pl.ds(ch * ZCH, ZCH): pltpu.make_async_copy(
                       znd.at[sl], acc.at[sl], isem).wait())
        plsc.subcore_barrier()

        def wave(w, pw, pre_pred, next_pred, first_pred):
            po = 1 - pw
            for q in range(WAVE):
                pltpu.make_async_copy(xs.at[idx_v.at[pw, 0, q]],
                                      rows_v.at[q % 2], gsem).wait()
                _maybe(first_pred if q == 0 else None,
                       lambda: pltpu.make_async_copy(
                           rows_v.at[(q + 1) % 2],
                           acc.at[idx_v.at[pw, 1, q]], ssem).wait())
                if q == 0:
                    _maybe(pre_pred, lambda: pltpu.async_copy(
                        e.at[:, pl.ds(base + (w + 1) * WAVE, WAVE)],
                        idx_v.at[po], isem))
                if q < WAVE - 1:
                    pltpu.async_copy(xs.at[idx_v.at[pw, 0, q + 1]],
                                     rows_v.at[(q + 1) % 2], gsem)
                if q == WAVE - 2:
                    _maybe(pre_pred, lambda: pltpu.make_async_copy(
                        e.at[:, pl.ds(base, WAVE)], idx_v.at[po], isem).wait())
                if q == WAVE - 1:
                    _maybe(next_pred, lambda: pltpu.async_copy(
                        xs.at[idx_v.at[po, 0, 0]], rows_v.at[0], gsem))
                pltpu.async_copy(rows_v.at[q % 2], acc.at[idx_v.at[pw, 1, q]],
                                 ssem, add=True)

        @pl.loop(0, IT_PT // 2)
        def pair(i):
            not_last = i < IT_PT // 2 - 1
            wave(2 * i, 0, None, None, i > 0)
            wave(2 * i + 1, 1, not_last, not_last, None)

        pltpu.make_async_copy(rows_v.at[1], acc.at[idx_v.at[1, 1, WAVE - 1]],
                              ssem).wait()

    pl.when(c == 0)(lambda: run(xs0, e0))
    pl.when(c == 1)(lambda: run(xs1, e1))
    plsc.subcore_barrier()

    def wb(y):
        for i in range(8):
            ch = t + NS * i
            _maybe(ch < NZCH,
                   lambda sl=pl.ds(ch * ZCH, ZCH): pltpu.async_copy(
                       acc.at[sl], y.at[sl], gsem))
        for i in range(8):
            ch = t + NS * i
            _maybe(ch < NZCH,
                   lambda sl=pl.ds(ch * ZCH, ZCH): pltpu.make_async_copy(
                       acc.at[sl], y.at[sl], gsem).wait())

    pl.when(c == 0)(lambda: wb(y0))
    pl.when(c == 1)(lambda: wb(y1))


_agg_call = pl.kernel(
    _agg_body,
    out_type=(jax.ShapeDtypeStruct((N, D), jnp.float32),
              jax.ShapeDtypeStruct((N, D), jnp.float32)),
    mesh=_mesh,
    scratch_types=[
        pltpu.VMEM_SHARED((N, D), jnp.float32),
        pltpu.VMEM((2, 2, WAVE, CH), jnp.int32),
        pltpu.VMEM((2, CH, D), jnp.float32),
        pltpu.SemaphoreType.DMA,
        pltpu.SemaphoreType.DMA,
        pltpu.SemaphoreType.DMA,
    ],
)


def _nrm(d):
    return jnp.where(d > 0, lax.rsqrt(jnp.maximum(d, 1.0)), 0.0)


def _norm_body(x_ref, od0, id0, od1, id1, xs0, xs1, ns0, ns1, nd0, nd1):
    a = _nrm(od0[...])
    b = _nrm(od1[...])
    xv = x_ref[...]
    xs0[...] = xv * a
    xs1[...] = xv * b
    ns0[...] = a
    ns1[...] = b
    nd0[...] = _nrm(id0[...])
    nd1[...] = _nrm(id1[...])


_BLK = 2000
_vec_spec = pl.BlockSpec((_BLK, 1), lambda i: (i, 0))
_mat_spec = pl.BlockSpec((_BLK, D), lambda i: (i, 0))

_norm_call = pl.pallas_call(
    _norm_body,
    grid=(N // _BLK,),
    in_specs=[_mat_spec, _vec_spec, _vec_spec, _vec_spec, _vec_spec],
    out_specs=[_mat_spec, _mat_spec, _vec_spec, _vec_spec, _vec_spec, _vec_spec],
    out_shape=[jax.ShapeDtypeStruct((N, D), jnp.float32),
               jax.ShapeDtypeStruct((N, D), jnp.float32),
               jax.ShapeDtypeStruct((N, 1), jnp.float32),
               jax.ShapeDtypeStruct((N, 1), jnp.float32),
               jax.ShapeDtypeStruct((N, 1), jnp.float32),
               jax.ShapeDtypeStruct((N, 1), jnp.float32)],
)


def _w_spec():
    return pl.BlockSpec((D, D), lambda i: (0, 0))


def _comb0_body(y0, y1, nd0, nd1, ns0, ns1, w0, w1, bs, hs0, hs1):
    h = jnp.dot(y0[...] * nd0[...], w0[...], preferred_element_type=jnp.float32)
    h = h + jnp.dot(y1[...] * nd1[...], w1[...], preferred_element_type=jnp.float32)
    h = jnp.maximum(h + bs[...], 0.0)
    hs0[...] = h * ns0[...]
    hs1[...] = h * ns1[...]


_comb0_call = pl.pallas_call(
    _comb0_body,
    grid=(N // _BLK,),
    in_specs=[_mat_spec, _mat_spec, _vec_spec, _vec_spec, _vec_spec, _vec_spec,
              _w_spec(), _w_spec(), pl.BlockSpec((1, D), lambda i: (0, 0))],
    out_specs=[_mat_spec, _mat_spec],
    out_shape=[jax.ShapeDtypeStruct((N, D), jnp.float32),
               jax.ShapeDtypeStruct((N, D), jnp.float32)],
)


def _comb1_body(y0, y1, nd0, nd1, w0, w1, bs, out):
    h = jnp.dot(y0[...] * nd0[...], w0[...], preferred_element_type=jnp.float32)
    h = h + jnp.dot(y1[...] * nd1[...], w1[...], preferred_element_type=jnp.float32)
    out[...] = h + bs[...]


_comb1_call = pl.pallas_call(
    _comb1_body,
    grid=(N // _BLK,),
    in_specs=[_mat_spec, _mat_spec, _vec_spec, _vec_spec,
              _w_spec(), _w_spec(), pl.BlockSpec((1, D), lambda i: (0, 0))],
    out_specs=_mat_spec,
    out_shape=jax.ShapeDtypeStruct((N, D), jnp.float32),
)


@jax.jit
def kernel(x, edge_index_r0, edge_index_r1, W0_r0, b0_r0, W0_r1, b0_r1,
           W1_r0, b1_r0, W1_r1, b1_r1):
    e0 = edge_index_r0.reshape(2, CROWS, CH)
    e1 = edge_index_r1.reshape(2, CROWS, CH)
    z_n = jnp.zeros((NPAD,), jnp.float32)
    znd = jnp.zeros((N, D), jnp.float32)

    og0, ig0, og1, ig1 = _deg_call(e0, e1, z_n)
    od0 = og0.reshape(NPAD, 1)
    id0 = ig0.reshape(NPAD, 1)
    od1 = og1.reshape(NPAD, 1)
    id1 = ig1.reshape(NPAD, 1)

    xs0, xs1, ns0, ns1, nd0, nd1 = _norm_call(x, od0, id0, od1, id1)
    y00, y01 = _agg_call(xs0, xs1, e0, e1, znd)
    bs0 = (b0_r0 + b0_r1).reshape(1, D)
    hs0, hs1 = _comb0_call(y00, y01, nd0, nd1, ns0, ns1, W0_r0, W0_r1, bs0)
    y10, y11 = _agg_call(hs0, hs1, e0, e1, znd)
    bs1 = (b1_r0 + b1_r1).reshape(1, D)
    return _comb1_call(y10, y11, nd0, nd1, W1_r0, W1_r1, bs1)
